# baseline (speedup 1.0000x reference)
import functools

import jax
import jax.numpy as jnp
import numpy as np
from jax import lax
from jax.experimental import pallas as pl
from jax.experimental.pallas import tpu as pltpu
from jax.experimental.pallas import tpu_sc as plsc

K = 32
EPS = 1e-5
NB = 200
QB = 400
CW = 512


def _proj_body(p_ref, x_ref, w1a_ref, w1b_ref, b1_ref, a_ref, p3_ref):
    p = p_ref[...]
    p3 = (p[:, 0:1] * w1a_ref[0:1, :]
          + p[:, 1:2] * w1a_ref[1:2, :]
          + p[:, 2:3] * w1a_ref[2:3, :])
    a = p3 + jnp.dot(x_ref[...], w1b_ref[...],
                     preferred_element_type=jnp.float32) + b1_ref[...]
    a_ref[...] = a
    p3_ref[...] = p3


def _stats1_body(acol_ref, p3_ref, acc_ref):
    t = acol_ref[...].reshape(NB, K, 128) - p3_ref[...][:, None, :]
    s = jnp.sum(t, axis=(0, 1))
    ss = jnp.sum(t * t, axis=(0, 1))
    upd = jnp.concatenate([s[None, :], ss[None, :],
                           jnp.zeros((6, 128), jnp.float32)], axis=0)

    @pl.when(pl.program_id(0) == 0)
    def _():
        acc_ref[...] = jnp.zeros_like(acc_ref)

    acc_ref[...] += upd


def _pass2_body(acol_ref, p3_ref, st1_ref, w2_ref, b2_ref, g1_ref, be1_ref,
                m_ref, acc_ref, inv_e):
    mu1 = st1_ref[0:1, :] * inv_e
    var1 = st1_ref[1:2, :] * inv_e - mu1 * mu1
    inv1 = g1_ref[...] * jax.lax.rsqrt(var1 + EPS)
    sh1 = be1_ref[...] - mu1 * inv1
    t = acol_ref[...].reshape(NB, K, 128) - p3_ref[...][:, None, :]
    u = jnp.maximum(t * inv1[None, :, :] + sh1[None, :, :], 0.0)
    h2 = jnp.dot(u.reshape(NB * K, 128), w2_ref[...],
                 preferred_element_type=jnp.float32) + b2_ref[...]
    s = jnp.sum(h2, axis=0)
    ss = jnp.sum(h2 * h2, axis=0)
    upd = jnp.concatenate([s[None, :], ss[None, :],
                           jnp.zeros((6, 128), jnp.float32)], axis=0)
    m_ref[...] = jnp.max(h2.reshape(NB, K, 128), axis=1)

    @pl.when(pl.program_id(0) == 0)
    def _():
        acc_ref[...] = jnp.zeros_like(acc_ref)

    acc_ref[...] += upd


def _final_body(m_ref, st2_ref, g2_ref, be2_ref, o_ref, inv_e):
    mu2 = st2_ref[0:1, :] * inv_e
    var2 = st2_ref[1:2, :] * inv_e - mu2 * mu2
    inv2 = g2_ref[...] * jax.lax.rsqrt(var2 + EPS)
    sh2 = be2_ref[...] - mu2 * inv2
    o_ref[...] = jnp.maximum(m_ref[...] * inv2 + sh2, 0.0)


def _knn_body(c0_ref, nc_ref, q_ref, bq_ref, ps_ref, bs_ref, col_ref, d2s):
    g = pl.program_id(0)
    c0 = c0_ref[g]
    nc = nc_ref[g]
    qx = q_ref[:, 0:1]
    qy = q_ref[:, 1:2]
    qz = q_ref[:, 2:3]
    bq = bq_ref[...]
    inf = jnp.float32(jnp.inf)
    bigi = jnp.int32(2 ** 30)
    liota = jax.lax.broadcasted_iota(jnp.int32, (1, CW), 1)

    def fill_chunk(j, _):
        cs = pl.multiple_of((c0 + j) * CW, CW)
        sx = ps_ref[0:1, pl.ds(cs, CW)]
        sy = ps_ref[1:2, pl.ds(cs, CW)]
        sz = ps_ref[2:3, pl.ds(cs, CW)]
        bs = bs_ref[0:1, pl.ds(cs, CW)]
        d2 = (qx - sx) ** 2 + (qy - sy) ** 2 + (qz - sz) ** 2
        d2s[:, pl.ds(j * CW, CW)] = jnp.where(bq != bs, inf, d2)
        return 0

    jax.lax.fori_loop(0, nc, fill_chunk, 0)

    prev = jnp.full((QB, 1), -1.0, jnp.float32)
    for it in range(K):
        def scan_chunk(j, carry, prev=prev):
            mv, mi = carry
            off = pl.multiple_of(j * CW, CW)
            lcol = liota + j * CW
            dd = d2s[:, pl.ds(off, CW)]
            t = jnp.where(dd > prev, dd, inf)
            cm = jnp.min(t, axis=1, keepdims=True)
            lfirst = jnp.min(jnp.where(t == cm, lcol, bigi), axis=1,
                             keepdims=True)
            upd = cm < mv
            return (jnp.where(upd, cm, mv), jnp.where(upd, lfirst, mi))

        mv0 = jnp.full((QB, 1), inf)
        mi0 = jnp.zeros((QB, 1), jnp.int32)
        prev, mi = jax.lax.fori_loop(0, nc, scan_chunk, (mv0, mi0))
        col_ref[:, it:it + 1] = mi + c0 * CW


def _knn_cols(p, b, k):
    n = p.shape[0]
    npad = ((n + CW - 1) // CW) * CW
    b32 = b.astype(jnp.int32)
    ps = jnp.zeros((8, npad), jnp.float32).at[:3, :n].set(p.T)
    bs = jnp.full((8, npad), -9, jnp.int32).at[0, :n].set(b32)
    bq = b32.reshape(n, 1)
    nblk = n // QB
    qlo = jnp.arange(nblk, dtype=jnp.int32) * QB
    starts = jnp.searchsorted(b32, jnp.arange(4, dtype=jnp.int32),
                              side='left').astype(jnp.int32)
    ends = jnp.searchsorted(b32, jnp.arange(4, dtype=jnp.int32),
                            side='right').astype(jnp.int32)
    sup_lo = starts[b32[qlo]]
    sup_hi = ends[b32[qlo + QB - 1]]
    chunk_lo = sup_lo // CW
    n_chunks = (sup_hi + CW - 1) // CW - chunk_lo
    col = pl.pallas_call(
        _knn_body,
        grid_spec=pltpu.PrefetchScalarGridSpec(
            num_scalar_prefetch=2,
            grid=(nblk,),
            in_specs=[
                pl.BlockSpec((QB, 3), lambda g, c0, nc: (g, 0)),
                pl.BlockSpec((QB, 1), lambda g, c0, nc: (g, 0)),
                pl.BlockSpec((8, npad), lambda g, c0, nc: (0, 0)),
                pl.BlockSpec((8, npad), lambda g, c0, nc: (0, 0)),
            ],
            out_specs=pl.BlockSpec((QB, k), lambda g, c0, nc: (g, 0)),
            scratch_shapes=[pltpu.VMEM((QB, npad), jnp.float32)],
        ),
        out_shape=jax.ShapeDtypeStruct((n, k), jnp.int32),
    )(chunk_lo, n_chunks, p, bq, ps, bs)
    return col.reshape(-1)


_SC_NW = 32
_SC_CB = 128


def _sc_gather(table, col):
    n, c = table.shape
    e = col.shape[0]
    nchunks = -(-e // (_SC_NW * _SC_CB))
    nchunks += nchunks % 2
    per_w = nchunks * _SC_CB
    epad = per_w * _SC_NW
    colp = jnp.zeros((epad,), jnp.int32).at[:e].set(col)
    mesh = plsc.VectorSubcoreMesh(core_axis_name="c", subcore_axis_name="s")

    @functools.partial(
        pl.kernel, mesh=mesh,
        out_type=jax.ShapeDtypeStruct((epad, c), jnp.float32),
        scratch_types=[
            pltpu.VMEM((_SC_CB,), jnp.int32),
            pltpu.VMEM((_SC_CB,), jnp.int32),
            pltpu.VMEM((_SC_CB, c), jnp.float32),
            pltpu.VMEM((_SC_CB, c), jnp.float32),
            pltpu.SemaphoreType.DMA,
            pltpu.SemaphoreType.DMA,
        ],
    )
    def gk(table_hbm, idx_hbm, out_hbm, idx_a, idx_b, rows_a, rows_b,
           sem_a, sem_b):
        wid = lax.axis_index("s") * 2 + lax.axis_index("c")
        wbase = wid * per_w

        def pair(jj, _):
            base_a = pl.multiple_of(wbase + jj * (2 * _SC_CB), _SC_CB)
            base_b = pl.multiple_of(base_a + _SC_CB, _SC_CB)
            pltpu.sync_copy(idx_hbm.at[pl.ds(base_a, _SC_CB)], idx_a)
            ha = pltpu.async_copy(table_hbm.at[idx_a], rows_a, sem_a)
            pltpu.sync_copy(idx_hbm.at[pl.ds(base_b, _SC_CB)], idx_b)
            hb = pltpu.async_copy(table_hbm.at[idx_b], rows_b, sem_b)
            ha.wait()
            pltpu.sync_copy(rows_a, out_hbm.at[pl.ds(base_a, _SC_CB)])
            hb.wait()
            pltpu.sync_copy(rows_b, out_hbm.at[pl.ds(base_b, _SC_CB)])
            return 0

        lax.fori_loop(0, nchunks // 2, pair, 0)

    return gk(table, colp)


def kernel(p, x, b, W1, bias1, g1, be1, W2, bias2, g2, be2):
    n, c = x.shape
    e_total = n * K
    inv_e = np.float32(1.0 / e_total)
    w1a = W1[:3]
    w1b = W1[3:]
    b1r = bias1.reshape(1, c)
    g1r = g1.reshape(1, c)
    be1r = be1.reshape(1, c)
    b2r = bias2.reshape(1, c)
    g2r = g2.reshape(1, c)
    be2r = be2.reshape(1, c)

    a, p3 = pl.pallas_call(
        _proj_body,
        out_shape=[jax.ShapeDtypeStruct((n, c), jnp.float32),
                   jax.ShapeDtypeStruct((n, c), jnp.float32)],
    )(p, x, w1a, w1b, b1r)

    col = _knn_cols(p, b, K)
    acol = _sc_gather(a, col)

    grid = n // NB
    st1 = pl.pallas_call(
        _stats1_body,
        grid=(grid,),
        in_specs=[pl.BlockSpec((NB * K, c), lambda i: (i, 0)),
                  pl.BlockSpec((NB, c), lambda i: (i, 0))],
        out_specs=pl.BlockSpec((8, c), lambda i: (0, 0)),
        out_shape=jax.ShapeDtypeStruct((8, c), jnp.float32),
    )(acol, p3)

    m, st2 = pl.pallas_call(
        functools.partial(_pass2_body, inv_e=inv_e),
        grid=(grid,),
        in_specs=[pl.BlockSpec((NB * K, c), lambda i: (i, 0)),
                  pl.BlockSpec((NB, c), lambda i: (i, 0)),
                  pl.BlockSpec((8, c), lambda i: (0, 0)),
                  pl.BlockSpec((c, c), lambda i: (0, 0)),
                  pl.BlockSpec((1, c), lambda i: (0, 0)),
                  pl.BlockSpec((1, c), lambda i: (0, 0)),
                  pl.BlockSpec((1, c), lambda i: (0, 0))],
        out_specs=[pl.BlockSpec((NB, c), lambda i: (i, 0)),
                   pl.BlockSpec((8, c), lambda i: (0, 0))],
        out_shape=[jax.ShapeDtypeStruct((n, c), jnp.float32),
                   jax.ShapeDtypeStruct((8, c), jnp.float32)],
    )(acol, p3, st1, W2, b2r, g1r, be1r)

    x_agg = pl.pallas_call(
        functools.partial(_final_body, inv_e=inv_e),
        out_shape=jax.ShapeDtypeStruct((n, c), jnp.float32),
    )(m, st2, g2r, be2r)

    return (p, x_agg, b)


# speedup vs baseline: 1.1349x; 1.1349x over previous
import functools

import jax
import jax.numpy as jnp
import numpy as np
from jax import lax
from jax.experimental import pallas as pl
from jax.experimental.pallas import tpu as pltpu
from jax.experimental.pallas import tpu_sc as plsc

K = 32
EPS = 1e-5
NB = 200
QB = 400
CW = 1024


def _proj_body(p_ref, x_ref, w1a_ref, w1b_ref, b1_ref, a_ref, p3_ref):
    p = p_ref[...]
    p3 = (p[:, 0:1] * w1a_ref[0:1, :]
          + p[:, 1:2] * w1a_ref[1:2, :]
          + p[:, 2:3] * w1a_ref[2:3, :])
    a = p3 + jnp.dot(x_ref[...], w1b_ref[...],
                     preferred_element_type=jnp.float32) + b1_ref[...]
    a_ref[...] = a
    p3_ref[...] = p3


def _stats1_body(acol_ref, p3_ref, acc_ref):
    t = acol_ref[...].reshape(NB, K, 128) - p3_ref[...][:, None, :]
    s = jnp.sum(t, axis=(0, 1))
    ss = jnp.sum(t * t, axis=(0, 1))
    upd = jnp.concatenate([s[None, :], ss[None, :],
                           jnp.zeros((6, 128), jnp.float32)], axis=0)

    @pl.when(pl.program_id(0) == 0)
    def _():
        acc_ref[...] = jnp.zeros_like(acc_ref)

    acc_ref[...] += upd


def _pass2_body(acol_ref, p3_ref, st1_ref, w2_ref, b2_ref, g1_ref, be1_ref,
                m_ref, acc_ref, inv_e):
    mu1 = st1_ref[0:1, :] * inv_e
    var1 = st1_ref[1:2, :] * inv_e - mu1 * mu1
    inv1 = g1_ref[...] * jax.lax.rsqrt(var1 + EPS)
    sh1 = be1_ref[...] - mu1 * inv1
    t = acol_ref[...].reshape(NB, K, 128) - p3_ref[...][:, None, :]
    u = jnp.maximum(t * inv1[None, :, :] + sh1[None, :, :], 0.0)
    h2 = jnp.dot(u.reshape(NB * K, 128), w2_ref[...],
                 preferred_element_type=jnp.float32) + b2_ref[...]
    s = jnp.sum(h2, axis=0)
    ss = jnp.sum(h2 * h2, axis=0)
    upd = jnp.concatenate([s[None, :], ss[None, :],
                           jnp.zeros((6, 128), jnp.float32)], axis=0)
    m_ref[...] = jnp.max(h2.reshape(NB, K, 128), axis=1)

    @pl.when(pl.program_id(0) == 0)
    def _():
        acc_ref[...] = jnp.zeros_like(acc_ref)

    acc_ref[...] += upd


def _final_body(m_ref, st2_ref, g2_ref, be2_ref, o_ref, inv_e):
    mu2 = st2_ref[0:1, :] * inv_e
    var2 = st2_ref[1:2, :] * inv_e - mu2 * mu2
    inv2 = g2_ref[...] * jax.lax.rsqrt(var2 + EPS)
    sh2 = be2_ref[...] - mu2 * inv2
    o_ref[...] = jnp.maximum(m_ref[...] * inv2 + sh2, 0.0)


def _knn_body(c0_ref, nc_ref, q_ref, bq_ref, ps_ref, bs_ref, col_ref, d2s):
    g = pl.program_id(0)
    c0 = c0_ref[g]
    nc = nc_ref[g]
    qx = q_ref[:, 0:1]
    qy = q_ref[:, 1:2]
    qz = q_ref[:, 2:3]
    bq = bq_ref[...]
    inf = jnp.float32(jnp.inf)
    bigi = jnp.int32(2 ** 30)
    liota = jax.lax.broadcasted_iota(jnp.int32, (1, CW), 1)

    def fill_chunk(j, _):
        cs = pl.multiple_of((c0 + j) * CW, CW)
        sx = ps_ref[0:1, pl.ds(cs, CW)]
        sy = ps_ref[1:2, pl.ds(cs, CW)]
        sz = ps_ref[2:3, pl.ds(cs, CW)]
        bs = bs_ref[0:1, pl.ds(cs, CW)]
        d2 = (qx - sx) ** 2 + (qy - sy) ** 2 + (qz - sz) ** 2
        d2s[:, pl.ds(j * CW, CW)] = jnp.where(bq != bs, inf, d2)
        return 0

    jax.lax.fori_loop(0, nc, fill_chunk, 0)

    prev = jnp.full((QB, 1), -1.0, jnp.float32)
    for it in range(K):
        def scan_chunk(j, carry, prev=prev):
            mv, mi = carry
            off = pl.multiple_of(j * CW, CW)
            lcol = liota + j * CW
            dd = d2s[:, pl.ds(off, CW)]
            t = jnp.where(dd > prev, dd, inf)
            cm = jnp.min(t, axis=1, keepdims=True)
            lfirst = jnp.min(jnp.where(t == cm, lcol, bigi), axis=1,
                             keepdims=True)
            upd = cm < mv
            return (jnp.where(upd, cm, mv), jnp.where(upd, lfirst, mi))

        mv0 = jnp.full((QB, 1), inf)
        mi0 = jnp.zeros((QB, 1), jnp.int32)
        prev, mi = jax.lax.fori_loop(0, nc, scan_chunk, (mv0, mi0))
        col_ref[:, it:it + 1] = mi + c0 * CW


def _knn_cols(p, b, k):
    n = p.shape[0]
    npad = ((n + CW - 1) // CW) * CW
    b32 = b.astype(jnp.int32)
    ps = jnp.zeros((8, npad), jnp.float32).at[:3, :n].set(p.T)
    bs = jnp.full((8, npad), -9, jnp.int32).at[0, :n].set(b32)
    bq = b32.reshape(n, 1)
    nblk = n // QB
    qlo = jnp.arange(nblk, dtype=jnp.int32) * QB
    starts = jnp.searchsorted(b32, jnp.arange(4, dtype=jnp.int32),
                              side='left').astype(jnp.int32)
    ends = jnp.searchsorted(b32, jnp.arange(4, dtype=jnp.int32),
                            side='right').astype(jnp.int32)
    sup_lo = starts[b32[qlo]]
    sup_hi = ends[b32[qlo + QB - 1]]
    chunk_lo = sup_lo // CW
    n_chunks = (sup_hi + CW - 1) // CW - chunk_lo
    col = pl.pallas_call(
        _knn_body,
        grid_spec=pltpu.PrefetchScalarGridSpec(
            num_scalar_prefetch=2,
            grid=(nblk,),
            in_specs=[
                pl.BlockSpec((QB, 3), lambda g, c0, nc: (g, 0)),
                pl.BlockSpec((QB, 1), lambda g, c0, nc: (g, 0)),
                pl.BlockSpec((8, npad), lambda g, c0, nc: (0, 0)),
                pl.BlockSpec((8, npad), lambda g, c0, nc: (0, 0)),
            ],
            out_specs=pl.BlockSpec((QB, k), lambda g, c0, nc: (g, 0)),
            scratch_shapes=[pltpu.VMEM((QB, npad), jnp.float32)],
        ),
        out_shape=jax.ShapeDtypeStruct((n, k), jnp.int32),
    )(chunk_lo, n_chunks, p, bq, ps, bs)
    return col.reshape(-1)


_SC_NW = 32
_SC_CB = 128


def _sc_gather(table, col):
    n, c = table.shape
    e = col.shape[0]
    nchunks = -(-e // (_SC_NW * _SC_CB))
    nchunks += nchunks % 2
    per_w = nchunks * _SC_CB
    epad = per_w * _SC_NW
    colp = jnp.zeros((epad,), jnp.int32).at[:e].set(col)
    mesh = plsc.VectorSubcoreMesh(core_axis_name="c", subcore_axis_name="s")

    @functools.partial(
        pl.kernel, mesh=mesh,
        out_type=jax.ShapeDtypeStruct((epad, c), jnp.float32),
        scratch_types=[
            pltpu.VMEM((_SC_CB,), jnp.int32),
            pltpu.VMEM((_SC_CB,), jnp.int32),
            pltpu.VMEM((_SC_CB, c), jnp.float32),
            pltpu.VMEM((_SC_CB, c), jnp.float32),
            pltpu.SemaphoreType.DMA,
            pltpu.SemaphoreType.DMA,
            pltpu.SemaphoreType.DMA,
            pltpu.SemaphoreType.DMA,
        ],
    )
    def gk(table_hbm, idx_hbm, out_hbm, idx_a, idx_b, rows_a, rows_b,
           sem_a, sem_b, sem_sa, sem_sb):
        wid = lax.axis_index("s") * 2 + lax.axis_index("c")
        wbase = wid * per_w
        w0 = pl.multiple_of(wbase, _SC_CB)
        w1 = pl.multiple_of(wbase + _SC_CB, _SC_CB)
        pltpu.async_copy(rows_a, out_hbm.at[pl.ds(w0, _SC_CB)], sem_sa)
        pltpu.async_copy(rows_b, out_hbm.at[pl.ds(w1, _SC_CB)], sem_sb)

        def pair(jj, _):
            base_a = pl.multiple_of(wbase + jj * (2 * _SC_CB), _SC_CB)
            base_b = pl.multiple_of(base_a + _SC_CB, _SC_CB)
            pltpu.make_async_copy(
                rows_a, out_hbm.at[pl.ds(base_a, _SC_CB)], sem_sa).wait()
            pltpu.sync_copy(idx_hbm.at[pl.ds(base_a, _SC_CB)], idx_a)
            ha = pltpu.async_copy(table_hbm.at[idx_a], rows_a, sem_a)
            pltpu.make_async_copy(
                rows_b, out_hbm.at[pl.ds(base_b, _SC_CB)], sem_sb).wait()
            pltpu.sync_copy(idx_hbm.at[pl.ds(base_b, _SC_CB)], idx_b)
            hb = pltpu.async_copy(table_hbm.at[idx_b], rows_b, sem_b)
            ha.wait()
            pltpu.async_copy(rows_a, out_hbm.at[pl.ds(base_a, _SC_CB)], sem_sa)
            hb.wait()
            pltpu.async_copy(rows_b, out_hbm.at[pl.ds(base_b, _SC_CB)], sem_sb)
            return 0

        lax.fori_loop(0, nchunks // 2, pair, 0)
        pltpu.make_async_copy(
            rows_a, out_hbm.at[pl.ds(w0, _SC_CB)], sem_sa).wait()
        pltpu.make_async_copy(
            rows_b, out_hbm.at[pl.ds(w1, _SC_CB)], sem_sb).wait()

    return gk(table, colp)


def kernel(p, x, b, W1, bias1, g1, be1, W2, bias2, g2, be2):
    n, c = x.shape
    e_total = n * K
    inv_e = np.float32(1.0 / e_total)
    w1a = W1[:3]
    w1b = W1[3:]
    b1r = bias1.reshape(1, c)
    g1r = g1.reshape(1, c)
    be1r = be1.reshape(1, c)
    b2r = bias2.reshape(1, c)
    g2r = g2.reshape(1, c)
    be2r = be2.reshape(1, c)

    a, p3 = pl.pallas_call(
        _proj_body,
        out_shape=[jax.ShapeDtypeStruct((n, c), jnp.float32),
                   jax.ShapeDtypeStruct((n, c), jnp.float32)],
    )(p, x, w1a, w1b, b1r)

    col = _knn_cols(p, b, K)
    acol = _sc_gather(a, col)

    grid = n // NB
    st1 = pl.pallas_call(
        _stats1_body,
        grid=(grid,),
        in_specs=[pl.BlockSpec((NB * K, c), lambda i: (i, 0)),
                  pl.BlockSpec((NB, c), lambda i: (i, 0))],
        out_specs=pl.BlockSpec((8, c), lambda i: (0, 0)),
        out_shape=jax.ShapeDtypeStruct((8, c), jnp.float32),
    )(acol, p3)

    m, st2 = pl.pallas_call(
        functools.partial(_pass2_body, inv_e=inv_e),
        grid=(grid,),
        in_specs=[pl.BlockSpec((NB * K, c), lambda i: (i, 0)),
                  pl.BlockSpec((NB, c), lambda i: (i, 0)),
                  pl.BlockSpec((8, c), lambda i: (0, 0)),
                  pl.BlockSpec((c, c), lambda i: (0, 0)),
                  pl.BlockSpec((1, c), lambda i: (0, 0)),
                  pl.BlockSpec((1, c), lambda i: (0, 0)),
                  pl.BlockSpec((1, c), lambda i: (0, 0))],
        out_specs=[pl.BlockSpec((NB, c), lambda i: (i, 0)),
                   pl.BlockSpec((8, c), lambda i: (0, 0))],
        out_shape=[jax.ShapeDtypeStruct((n, c), jnp.float32),
                   jax.ShapeDtypeStruct((8, c), jnp.float32)],
    )(acol, p3, st1, W2, b2r, g1r, be1r)

    x_agg = pl.pallas_call(
        functools.partial(_final_body, inv_e=inv_e),
        out_shape=jax.ShapeDtypeStruct((n, c), jnp.float32),
    )(m, st2, g2r, be2r)

    return (p, x_agg, b)
